# initial kernel scaffold (unmeasured)
import jax
import jax.numpy as jnp
from jax import lax
from jax.experimental import pallas as pl
from jax.experimental.pallas import tpu as pltpu


def kernel(
    x,
):
    def body(*refs):
        pass

    out_shape = jax.ShapeDtypeStruct(..., jnp.float32)
    return pl.pallas_call(body, out_shape=out_shape)(...)


# baseline (device time: 221113 ns/iter reference)
import jax
import jax.numpy as jnp
from jax import lax
from jax.experimental import pallas as pl
from jax.experimental.pallas import tpu as pltpu

K = 16


def kernel(x):
    m, n = x.shape
    cm = m // K
    xb = x.astype(jnp.bfloat16)

    def body(x_any, x_blk, out_blk, comm_ref, send_sems, recv_sems):
        i = pl.program_id(0)
        my_x = lax.axis_index("x")
        my_y = lax.axis_index("y")
        my_z = lax.axis_index("z")
        peer = (1 - my_x, my_y, my_z)

        @pl.when(i == 0)
        def _():
            barrier_sem = pltpu.get_barrier_semaphore()
            pl.semaphore_signal(
                barrier_sem, inc=1, device_id=peer,
                device_id_type=pl.DeviceIdType.MESH,
            )
            pl.semaphore_wait(barrier_sem, 1)
            for k in range(K):
                sl = pl.ds(k * cm, cm)
                pltpu.make_async_remote_copy(
                    src_ref=x_any.at[sl],
                    dst_ref=comm_ref.at[sl],
                    send_sem=send_sems.at[k],
                    recv_sem=recv_sems.at[k],
                    device_id=peer,
                    device_id_type=pl.DeviceIdType.MESH,
                ).start()

        sl = pl.ds(i * cm, cm)
        pltpu.make_async_remote_copy(
            src_ref=x_any.at[sl],
            dst_ref=comm_ref.at[sl],
            send_sem=send_sems.at[0],
            recv_sem=recv_sems.at[i],
            device_id=peer,
            device_id_type=pl.DeviceIdType.MESH,
        ).wait_recv()

        out_blk[:, :] = x_blk[:, :].astype(jnp.float32) + comm_ref[
            sl, :
        ].astype(jnp.float32)

        @pl.when(i == K - 1)
        def _():
            for k in range(K):
                sl2 = pl.ds(k * cm, cm)
                pltpu.make_async_remote_copy(
                    src_ref=x_any.at[sl2],
                    dst_ref=comm_ref.at[sl2],
                    send_sem=send_sems.at[k],
                    recv_sem=recv_sems.at[0],
                    device_id=peer,
                    device_id_type=pl.DeviceIdType.MESH,
                ).wait_send()

    return pl.pallas_call(
        body,
        grid=(K,),
        out_shape=jax.ShapeDtypeStruct((m, n), jnp.float32),
        in_specs=[
            pl.BlockSpec(memory_space=pl.ANY),
            pl.BlockSpec((cm, n), lambda i: (i, 0)),
        ],
        out_specs=pl.BlockSpec((cm, n), lambda i: (i, 0)),
        scratch_shapes=[
            pltpu.VMEM((m, n), jnp.bfloat16),
            pltpu.SemaphoreType.DMA((K,)),
            pltpu.SemaphoreType.DMA((K,)),
        ],
        compiler_params=pltpu.CompilerParams(
            collective_id=0,
            dimension_semantics=("arbitrary",),
        ),
    )(xb, xb)


# device time: 145128 ns/iter; 1.5236x vs baseline; 1.5236x over previous
import jax
import jax.numpy as jnp
from jax import lax
from jax.experimental import pallas as pl
from jax.experimental.pallas import tpu as pltpu

KH = 8


def kernel(x):
    m, n = x.shape
    half = m // 2
    ch = half // KH
    xb = x.astype(jnp.bfloat16)

    def body(
        x_any,
        out_any,
        commA,
        commB,
        sendB,
        xloc,
        stage,
        semA_send,
        semA_recv,
        semB_send,
        semB_recv,
        xloc_sem,
        stage_sem,
    ):
        my_x = lax.axis_index("x")
        my_y = lax.axis_index("y")
        my_z = lax.axis_index("z")
        xpeer = (1 - my_x, my_y, my_z)
        ypeer = (my_x, 1 - my_y, my_z)

        barrier_sem = pltpu.get_barrier_semaphore()
        for p in (xpeer, ypeer):
            pl.semaphore_signal(
                barrier_sem, inc=1, device_id=p,
                device_id_type=pl.DeviceIdType.MESH,
            )
        pl.semaphore_wait(barrier_sem, 2)

        def inner(hoff, ooff):
            def rdmaA(k):
                return pltpu.make_async_remote_copy(
                    src_ref=x_any.at[pl.ds(hoff + k * ch, ch)],
                    dst_ref=commA.at[pl.ds(k * ch, ch)],
                    send_sem=semA_send.at[k],
                    recv_sem=semA_recv.at[k],
                    device_id=xpeer,
                    device_id_type=pl.DeviceIdType.MESH,
                )

            def rdmaB(k):
                return pltpu.make_async_remote_copy(
                    src_ref=sendB.at[pl.ds(k * ch, ch)],
                    dst_ref=commB.at[pl.ds(k * ch, ch)],
                    send_sem=semB_send.at[k],
                    recv_sem=semB_recv.at[k],
                    device_id=ypeer,
                    device_id_type=pl.DeviceIdType.MESH,
                )

            def xloc_copy(k):
                return pltpu.make_async_copy(
                    x_any.at[pl.ds(hoff + k * ch, ch)],
                    xloc.at[k % 2],
                    xloc_sem.at[k % 2],
                )

            def stage_copy(slot, row_off):
                return pltpu.make_async_copy(
                    stage.at[slot],
                    out_any.at[pl.ds(row_off, ch)],
                    stage_sem.at[slot],
                )

            for k in range(KH):
                rdmaA(k).start()
            xloc_copy(0).start()
            xloc_copy(1).start()

            stage_rows = [None, None]
            t = 0

            for k in range(KH):
                xloc_copy(k).wait()
                rdmaA(k).wait_recv()
                s32 = xloc[k % 2].astype(jnp.float32) + commA[
                    pl.ds(k * ch, ch), :
                ].astype(jnp.float32)
                sendB[pl.ds(k * ch, ch), :] = s32.astype(jnp.bfloat16)
                rdmaB(k).start()
                slot = t % 2
                if stage_rows[slot] is not None:
                    stage_copy(slot, stage_rows[slot]).wait()
                stage[slot] = s32
                stage_rows[slot] = hoff + k * ch
                stage_copy(slot, stage_rows[slot]).start()
                t += 1
                if k + 2 < KH:
                    xloc_copy(k + 2).start()

            for j in range(KH):
                rdmaB(j).wait_recv()
                s = commB[pl.ds(j * ch, ch), :].astype(jnp.float32)
                slot = t % 2
                if stage_rows[slot] is not None:
                    stage_copy(slot, stage_rows[slot]).wait()
                stage[slot] = s
                stage_rows[slot] = ooff + j * ch
                stage_copy(slot, stage_rows[slot]).start()
                t += 1

            for k in range(KH):
                rdmaA(k).wait_send()
                rdmaB(k).wait_send()
            for slot in range(2):
                stage_copy(slot, stage_rows[slot]).wait()

        @pl.when(my_y == 0)
        def _():
            inner(0, half)

        @pl.when(my_y == 1)
        def _():
            inner(half, 0)

    return pl.pallas_call(
        body,
        out_shape=jax.ShapeDtypeStruct((m, n), jnp.float32),
        in_specs=[pl.BlockSpec(memory_space=pl.ANY)],
        out_specs=pl.BlockSpec(memory_space=pl.ANY),
        scratch_shapes=[
            pltpu.VMEM((half, n), jnp.bfloat16),
            pltpu.VMEM((half, n), jnp.bfloat16),
            pltpu.VMEM((half, n), jnp.bfloat16),
            pltpu.VMEM((2, ch, n), jnp.bfloat16),
            pltpu.VMEM((2, ch, n), jnp.float32),
            pltpu.SemaphoreType.DMA((KH,)),
            pltpu.SemaphoreType.DMA((KH,)),
            pltpu.SemaphoreType.DMA((KH,)),
            pltpu.SemaphoreType.DMA((KH,)),
            pltpu.SemaphoreType.DMA((2,)),
            pltpu.SemaphoreType.DMA((2,)),
        ],
        compiler_params=pltpu.CompilerParams(collective_id=0),
    )(xb)


# device time: 109365 ns/iter; 2.0218x vs baseline; 1.3270x over previous
import jax
import jax.numpy as jnp
from jax import lax
from jax.experimental import pallas as pl
from jax.experimental.pallas import tpu as pltpu

KH = 16
D = 4
RA = 8
RB = 8
RX = 6


def kernel(x):
    m, n = x.shape
    half = m // 2
    ch = half // KH
    assert RX > D + 1 and RA > D and RB > 1

    def body(
        x_any,
        out_any,
        commA,
        commB,
        sendA,
        sendB,
        xf32,
        semA_send,
        semA_recv,
        semB_send,
        semB_recv,
        xf32_sem,
        outA_sem,
        outB_sem,
    ):
        my_x = lax.axis_index("x")
        my_y = lax.axis_index("y")
        my_z = lax.axis_index("z")
        xpeer = (1 - my_x, my_y, my_z)
        ypeer = (my_x, 1 - my_y, my_z)

        barrier_sem = pltpu.get_barrier_semaphore()
        for p in (xpeer, ypeer):
            pl.semaphore_signal(
                barrier_sem, inc=1, device_id=p,
                device_id_type=pl.DeviceIdType.MESH,
            )
        pl.semaphore_wait(barrier_sem, 2)

        def inner(hoff, ooff):
            def rdmaA(k):
                return pltpu.make_async_remote_copy(
                    src_ref=sendA.at[k % RA],
                    dst_ref=commA.at[pl.ds(k * ch, ch)],
                    send_sem=semA_send.at[k],
                    recv_sem=semA_recv.at[k],
                    device_id=xpeer,
                    device_id_type=pl.DeviceIdType.MESH,
                )

            def rdmaB(k):
                return pltpu.make_async_remote_copy(
                    src_ref=sendB.at[k % RB],
                    dst_ref=commB.at[pl.ds(k * ch, ch)],
                    send_sem=semB_send.at[k],
                    recv_sem=semB_recv.at[k],
                    device_id=ypeer,
                    device_id_type=pl.DeviceIdType.MESH,
                )

            def xf32_copy(k):
                return pltpu.make_async_copy(
                    x_any.at[pl.ds(hoff + k * ch, ch)],
                    xf32.at[k % RX],
                    xf32_sem.at[k % RX],
                )

            def outA_copy(k):
                return pltpu.make_async_copy(
                    sendB.at[k % RB],
                    out_any.at[pl.ds(hoff + k * ch, ch)],
                    outA_sem.at[k],
                )

            def outB_copy(j):
                return pltpu.make_async_copy(
                    commB.at[pl.ds(j * ch, ch)],
                    out_any.at[pl.ds(ooff + j * ch, ch)],
                    outB_sem.at[j],
                )

            def feed(j):
                xf32_copy(j).wait()
                if j >= RA:
                    rdmaA(j - RA).wait_send()
                sendA[j % RA] = xf32[j % RX].astype(jnp.bfloat16)
                rdmaA(j).start()
                nxt = j + 2
                if RX <= nxt < KH:
                    xf32_copy(nxt).start()

            def consume(k):
                rdmaA(k).wait_recv()
                if k >= RB:
                    rdmaB(k - RB).wait_send()
                    outA_copy(k - RB).wait()
                sendB[k % RB] = (
                    sendA[k % RA].astype(jnp.float32)
                    + commA[pl.ds(k * ch, ch), :].astype(jnp.float32)
                ).astype(jnp.bfloat16)
                rdmaB(k).start()
                outA_copy(k).start()

            for j in range(RX):
                xf32_copy(j).start()
            for j in range(D):
                feed(j)

            for k in range(KH):
                if k + D < KH:
                    feed(k + D)
                consume(k)

            for j in range(KH):
                rdmaB(j).wait_recv()
                outB_copy(j).start()

            for k in range(KH - RA, KH):
                rdmaA(k).wait_send()
            for k in range(KH - RB, KH):
                rdmaB(k).wait_send()
                outA_copy(k).wait()
            for j in range(KH):
                outB_copy(j).wait()

        @pl.when(my_y == 0)
        def _():
            inner(0, half)

        @pl.when(my_y == 1)
        def _():
            inner(half, 0)

    return pl.pallas_call(
        body,
        out_shape=jax.ShapeDtypeStruct((m, n), jnp.bfloat16),
        in_specs=[pl.BlockSpec(memory_space=pl.ANY)],
        out_specs=pl.BlockSpec(memory_space=pl.ANY),
        scratch_shapes=[
            pltpu.VMEM((half, n), jnp.bfloat16),
            pltpu.VMEM((half, n), jnp.bfloat16),
            pltpu.VMEM((RA, ch, n), jnp.bfloat16),
            pltpu.VMEM((RB, ch, n), jnp.bfloat16),
            pltpu.VMEM((RX, ch, n), jnp.float32),
            pltpu.SemaphoreType.DMA((KH,)),
            pltpu.SemaphoreType.DMA((KH,)),
            pltpu.SemaphoreType.DMA((KH,)),
            pltpu.SemaphoreType.DMA((KH,)),
            pltpu.SemaphoreType.DMA((RX,)),
            pltpu.SemaphoreType.DMA((KH,)),
            pltpu.SemaphoreType.DMA((KH,)),
        ],
        compiler_params=pltpu.CompilerParams(collective_id=0),
    )(x)
